# baseline (speedup 1.0000x reference)
import functools

import jax
import jax.numpy as jnp
from jax.experimental import pallas as pl
from jax.experimental.pallas import tpu as pltpu
from jax.experimental.pallas import tpu_sc as plsc

_E = 8
_TOPK = 2
_L = 16
_NW = 32


def _pool_kernel(x_ref, out_ref, xb_ref):
    ci = pl.program_id(1)
    w = x_ref.shape[3]
    scale = 1.0 / (w * w)
    xv = x_ref[0]
    s = (jnp.sum(xv, axis=(1, 2)) * scale).reshape(-1, 1)

    @pl.when(ci == 0)
    def _():
        out_ref[0] = s

    @pl.when(ci > 0)
    def _():
        out_ref[0] = out_ref[0] + s

    xb_ref[0] = xv.astype(jnp.bfloat16)


def _sc_gating_kernel(gx_hbm, wg_hbm, wf_hbm,
                      wcomb_hbm, loss_hbm,
                      gx_v, wg_v, w_v, acc_v, loss_v,
                      B, IC, CPT):
    wid = jax.lax.axis_index("s") * 2 + jax.lax.axis_index("c")
    lane = jax.lax.iota(jnp.int32, _L)
    emask = lane < _E

    gd = jax.lax.GatherDimensionNumbers(
        offset_dims=(), collapsed_slice_dims=(0,), start_index_map=(0,))

    def _allred(v, op):
        r = v
        for s in (8, 4, 2, 1):
            idx = jnp.bitwise_and(lane + s, _L - 1)
            perm = jax.lax.gather(
                r, idx[:, None], gd, (1,),
                mode=jax.lax.GatherScatterMode.PROMISE_IN_BOUNDS)
            r = op(r, perm)
        return r

    pltpu.sync_copy(gx_hbm, gx_v)
    pltpu.sync_copy(wg_hbm, wg_v)

    gates_list = []
    for b in range(B):
        def body(j, acc):
            gxc = gx_v[b, pl.ds(j * _L, _L)]
            for l in range(_L):
                acc = acc + gxc[l] * wg_v[j * _L + l, :]
            return acc
        logits = jax.lax.fori_loop(0, IC // _L, body,
                                   jnp.zeros((_L,), jnp.float32))
        z = jnp.where(emask, logits, -jnp.inf)
        m = _allred(z, jnp.maximum)
        ez = jnp.where(emask, jnp.exp(z - m), 0.0)
        p = ez / _allred(ez, jnp.add)
        m1 = _allred(p, jnp.maximum)
        e1 = _allred(jnp.where(p == m1, lane, _E), jnp.minimum)
        p2 = jnp.where(lane == e1, -jnp.inf, jnp.where(emask, p, -jnp.inf))
        m2 = _allred(p2, jnp.maximum)
        e2 = _allred(jnp.where(p2 == m2, lane, _E), jnp.minimum)
        denom = m1 + m2 + 1e-6
        gates_b = (jnp.where(lane == e1, m1, 0.0)
                   + jnp.where(lane == e2, m2, 0.0)) / denom
        gates_list.append(gates_b)

    @pl.when(wid == 0)
    def _():
        imp = jnp.zeros((_L,), jnp.float32)
        for b in range(B):
            imp = imp + gates_list[b]
        ld = jnp.zeros((_L,), jnp.float32)
        for b in range(B):
            ld = ld + jnp.where(gates_list[b] > 0, 1.0, 0.0)

        def cv_sq(v):
            mean = _allred(jnp.where(emask, v, 0.0), jnp.add) / _E
            d = jnp.where(emask, v - mean, 0.0)
            var = _allred(d * d, jnp.add) / (_E - 1)
            return var / (mean * mean + 1e-10)

        loss = (cv_sq(imp) + cv_sq(ld)) * 0.01
        loss_v[:] = jnp.where(lane == 0, loss, 0.0)
        pltpu.sync_copy(loss_v, loss_hbm)

    base = wid * CPT
    pltpu.sync_copy(wf_hbm.at[:, pl.ds(base, CPT)], w_v)
    nch = CPT // _L
    for b in range(B):
        gb = [gates_list[b][e] for e in range(_E)]

        def wbody(j, _):
            s = pl.ds(j * _L, _L)
            a = gb[0] * w_v[0, s]
            for e in range(1, _E):
                a = a + gb[e] * w_v[e, s]
            acc_v[b, s] = a
            return 0

        jax.lax.fori_loop(0, nch, wbody, 0)
    pltpu.sync_copy(acc_v, wcomb_hbm.at[:, pl.ds(base, CPT)])


def _shift_col(p):
    return jnp.concatenate(
        [jnp.zeros((p.shape[0], 1, p.shape[2]), p.dtype), p[:, :-1, :]],
        axis=1)


def _shift_row(p, prev_block):
    ci = pl.program_id(1)
    prev_row = prev_block[-1:, :, :]
    prev_row = jnp.where(ci == 0, jnp.zeros_like(prev_row), prev_row)
    return jnp.concatenate([prev_row, p[:-1, :, :]], axis=0)


def _conv_kernel(w_ref, p00_ref, p01_ref, p10_ref, p11_ref,
                 p10h_ref, p11h_ref, b_ref, out_ref):
    R, OW, OC = p00_ref.shape[3], p00_ref.shape[4], out_ref.shape[1]
    p00 = p00_ref[0, 0, 0]
    p01 = p01_ref[0, 0, 0]
    p10 = p10_ref[0, 0, 0]
    p11 = p11_ref[0, 0, 0]
    p10m = _shift_row(p10, p10h_ref[0, 0, 0])
    p11m = _shift_row(p11, p11h_ref[0, 0, 0])
    taps = (
        (_shift_col(p11m), 0), (p10m, 1), (p11m, 2),
        (_shift_col(p01), 3), (p00, 4), (p01, 5),
        (_shift_col(p11), 6), (p10, 7), (p11, 8),
    )
    acc = None
    for patch, t in taps:
        patch = patch.reshape(R * OW, patch.shape[2])
        d = jnp.dot(patch, w_ref[0, t], preferred_element_type=jnp.float32)
        acc = d if acc is None else acc + d
    res = (acc + b_ref[0]).reshape(R, OW, OC)
    out_ref[0] = jnp.transpose(res, (2, 0, 1))


def kernel(x, train, w_gate, w_noise, W, b):
    del train, w_noise
    B, IC, H, Wd = x.shape
    E, OC = W.shape[0], W.shape[1]
    OH, OW = H // 2, Wd // 2

    hchunks = 4
    HB = H // hchunks
    gate_x, xb = pl.pallas_call(
        _pool_kernel,
        grid=(B, hchunks),
        in_specs=[pl.BlockSpec((1, IC, HB, Wd),
                               lambda bi, ci: (bi, 0, ci, 0))],
        out_specs=(
            pl.BlockSpec((1, IC, 1), lambda bi, ci: (bi, 0, 0)),
            pl.BlockSpec((1, IC, HB, Wd), lambda bi, ci: (bi, 0, ci, 0)),
        ),
        out_shape=(
            jax.ShapeDtypeStruct((B, IC, 1), jnp.float32),
            jax.ShapeDtypeStruct((B, IC, H, Wd), jnp.bfloat16),
        ),
        compiler_params=pltpu.CompilerParams(
            dimension_semantics=("arbitrary", "arbitrary")),
    )(x)
    gate_x = gate_x.reshape(B, IC)

    NW9 = 9 * IC * OC
    W_flat = jnp.transpose(W, (0, 3, 4, 2, 1)).reshape(E, NW9)
    wb = jnp.concatenate([W_flat, b], axis=1)
    CPT = -(-(NW9 + OC) // (_NW * 128)) * 128
    TOT = _NW * CPT
    wb = jnp.pad(wb, ((0, 0), (0, TOT - NW9 - OC)))
    wg_pad = jnp.pad(w_gate, ((0, 0), (0, _L - E)))

    mesh = plsc.VectorSubcoreMesh(core_axis_name="c", subcore_axis_name="s")
    sc_gate = pl.kernel(
        functools.partial(_sc_gating_kernel, B=B, IC=IC, CPT=CPT),
        mesh=mesh,
        out_type=(
            jax.ShapeDtypeStruct((B, TOT), jnp.float32),
            jax.ShapeDtypeStruct((_L,), jnp.float32),
        ),
        scratch_types=[
            pltpu.VMEM((B, IC), jnp.float32),
            pltpu.VMEM((IC, _L), jnp.float32),
            pltpu.VMEM((E, CPT), jnp.float32),
            pltpu.VMEM((B, CPT), jnp.float32),
            pltpu.VMEM((_L,), jnp.float32),
        ],
    )
    wcomb_ext, loss16 = sc_gate(gate_x, wg_pad, wb)
    w_comb = wcomb_ext[:, :NW9].astype(jnp.bfloat16).reshape(B, 9, IC, OC)
    b_comb = wcomb_ext[:, NW9:NW9 + OC].reshape(B, 1, OC)
    loss = loss16[:1].reshape(1, 1)

    xr = xb.reshape(B, IC, OH, 2, OW, 2).transpose(0, 3, 5, 2, 4, 1)

    rchunks = 7
    R = OH // rchunks
    blk = (1, 1, 1, R, OW, IC)

    def _phase(ry, rx):
        return pl.BlockSpec(blk, lambda bi, ci: (bi, ry, rx, ci, 0, 0))

    def _halo(ry, rx):
        return pl.BlockSpec(
            blk, lambda bi, ci: (bi, ry, rx, jnp.maximum(ci - 1, 0), 0, 0))

    y = pl.pallas_call(
        _conv_kernel,
        grid=(B, rchunks),
        in_specs=[
            pl.BlockSpec((1, 9, IC, OC), lambda bi, ci: (bi, 0, 0, 0)),
            _phase(0, 0), _phase(0, 1), _phase(1, 0), _phase(1, 1),
            _halo(1, 0), _halo(1, 1),
            pl.BlockSpec((1, 1, OC), lambda bi, ci: (bi, 0, 0)),
        ],
        out_specs=pl.BlockSpec((1, OC, R, OW), lambda bi, ci: (bi, 0, ci, 0)),
        out_shape=jax.ShapeDtypeStruct((B, OC, OH, OW), jnp.float32),
        compiler_params=pltpu.CompilerParams(
            dimension_semantics=("parallel", "arbitrary")),
    )(w_comb, xr, xr, xr, xr, xr, xr, b_comb)

    return y, loss.reshape(())
